# Initial kernel scaffold; baseline (speedup 1.0000x reference)
import jax
import jax.numpy as jnp
from jax.experimental import pallas as pl


def kernel(x_in, x_ind_in):
    raise NotImplementedError("write your pallas kernel here")


# speedup vs baseline: 2.0179x; 2.0179x over previous
import jax
import jax.numpy as jnp
from jax import lax
from jax.experimental import pallas as pl
from jax.experimental.pallas import tpu as pltpu
from jax.experimental.pallas import tpu_sc as plsc

N, H, W, C = 8, 112, 112, 96
OUT_SHAPE = (N, 2 * H, 2 * W, C)
LIN = N * 2 * H * 2 * W * C
NPAIRS = N * H * W * C

NC, NS, L = 2, 16, 16
CH = 1_835_008
NCHUNK = 21
NCH_SC = (NCHUNK + 1) // 2

PAIRS_PER_TILE = NPAIRS // NS
WIN = 4096
NWIN = PAIRS_PER_TILE // WIN
NVEC = WIN // L
SROW = 128
NROW = 17
FLUSH = (NROW - 1) * SROW
TSLICE = CH // NS


def _body(idx_hbm, val_hbm, zeros_hbm, out_hbm, idx_win, val_win, st_idx,
          st_val, chunk):
    c = lax.axis_index("c")
    s = lax.axis_index("s")
    pair_base = s * PAIRS_PER_TILE
    zslice = s * TSLICE
    zf = jnp.zeros((L,), jnp.float32)
    zi = jnp.zeros((L,), jnp.int32)
    one_i = jnp.ones((L,), jnp.int32)
    ch_u = jnp.full((L,), CH, jnp.uint32)

    for r in range(NROW):
        for z in range(SROW // L):
            st_idx[r, pl.ds(z * L, L)] = zi
            st_val[r, pl.ds(z * L, L)] = zf

    def chunk_body(p, _):
        chunk_id = 2 * p + c
        lo = chunk_id * CH
        lo_vec = jnp.full((L,), lo, jnp.int32)
        live = chunk_id < NCHUNK

        @pl.when(live)
        def _process():
            pltpu.sync_copy(zeros_hbm, chunk.at[pl.ds(zslice, TSLICE)])
            plsc.subcore_barrier()

            def win_body(w, cnt):
                woff = pair_base + w * WIN
                pltpu.sync_copy(idx_hbm.at[pl.ds(woff, WIN)], idx_win)
                pltpu.sync_copy(val_hbm.at[pl.ds(woff, WIN)], val_win)

                def vec_body(i, cnt):
                    iv = idx_win[pl.ds(i * L, L)]
                    vv = val_win[pl.ds(i * L, L)]
                    u = iv - lo_vec
                    m = plsc.bitcast(u, jnp.uint32) < ch_u
                    mi = jnp.where(m, one_i, zi)
                    pos = cnt + plsc.cumsum(mi) - mi
                    plsc.store_scatter(st_idx, [pos >> 7, pos & (SROW - 1)],
                                       u, mask=m)
                    plsc.store_scatter(st_val, [pos >> 7, pos & (SROW - 1)],
                                       vv, mask=m)
                    cnt = cnt + jnp.sum(mi)
                    full = cnt >= FLUSH

                    @pl.when(full)
                    def _flush():
                        for r in range(NROW):
                            pltpu.sync_copy(st_val.at[r],
                                            chunk.at[st_idx.at[r]], add=True)
                            for z in range(SROW // L):
                                st_val[r, pl.ds(z * L, L)] = zf

                    return jnp.where(full, 0, cnt)

                return lax.fori_loop(0, NVEC, vec_body, cnt)

            lax.fori_loop(0, NWIN, win_body, 0)

            for r in range(NROW):
                pltpu.sync_copy(st_val.at[r], chunk.at[st_idx.at[r]], add=True)
                for z in range(SROW // L):
                    st_val[r, pl.ds(z * L, L)] = zf

            plsc.subcore_barrier()
            pltpu.sync_copy(chunk.at[pl.ds(zslice, TSLICE)],
                            out_hbm.at[pl.ds(lo + zslice, TSLICE)])

        return 0

    lax.fori_loop(0, NCH_SC, chunk_body, 0)


_scatter_add = pl.kernel(
    _body,
    out_type=jax.ShapeDtypeStruct((LIN,), jnp.float32),
    mesh=plsc.VectorSubcoreMesh(core_axis_name="c", subcore_axis_name="s"),
    compiler_params=pltpu.CompilerParams(needs_layout_passes=False),
    scratch_types=[
        pltpu.VMEM((WIN,), jnp.int32),
        pltpu.VMEM((WIN,), jnp.float32),
        pltpu.VMEM((NROW, SROW), jnp.int32),
        pltpu.VMEM((NROW, SROW), jnp.float32),
        pltpu.VMEM_SHARED((CH,), jnp.float32),
    ],
)


@jax.jit
def kernel(x_in, x_ind_in):
    flat_idx = x_ind_in.reshape(-1)
    flat_val = x_in.reshape(-1)
    zeros = jnp.zeros((TSLICE,), jnp.float32)
    out = _scatter_add(flat_idx, flat_val, zeros)
    return out.reshape(OUT_SHAPE)


# speedup vs baseline: 4.5061x; 2.2331x over previous
import jax
import jax.numpy as jnp
from jax import lax
from jax.experimental import pallas as pl
from jax.experimental.pallas import tpu as pltpu
from jax.experimental.pallas import tpu_sc as plsc

N, H, W, C = 8, 112, 112, 96
OUT_SHAPE = (N, 2 * H, 2 * W, C)
LIN = N * 2 * H * 2 * W * C
NPAIRS = N * H * W * C

NC, NS, L = 2, 16, 16
CH = 1_835_008
NCHUNK = 21
NCH_SC = (NCHUNK + 1) // 2

PAIRS_PER_TILE = NPAIRS // NS
WIN = 2048
NWIN = PAIRS_PER_TILE // WIN
NVEC = WIN // L
SROW = 128
NROW = WIN // SROW + 2
TSLICE = CH // NS


def _body(idx_hbm, val_hbm, zeros_hbm, out_hbm, idx_win, val_win, st_idx,
          st_val, chunk, sem_i0, sem_v0, sem_i1, sem_v1):
    c = lax.axis_index("c")
    s = lax.axis_index("s")
    pair_base = s * PAIRS_PER_TILE
    zslice = s * TSLICE
    zf = jnp.zeros((L,), jnp.float32)
    zi = jnp.zeros((L,), jnp.int32)
    one_i = jnp.ones((L,), jnp.int32)
    ch_u = jnp.full((L,), CH, jnp.uint32)
    sems = ((sem_i0, sem_v0), (sem_i1, sem_v1))

    for r in range(NROW):
        for z in range(SROW // L):
            st_idx[r, pl.ds(z * L, L)] = zi
            st_val[r, pl.ds(z * L, L)] = zf

    def _start(w, b):
        woff = pair_base + w * WIN
        pltpu.async_copy(idx_hbm.at[pl.ds(woff, WIN)], idx_win.at[b],
                         sems[b][0])
        pltpu.async_copy(val_hbm.at[pl.ds(woff, WIN)], val_win.at[b],
                         sems[b][1])

    def _wait(w, b):
        woff = pair_base + w * WIN
        pltpu.make_async_copy(idx_hbm.at[pl.ds(woff, WIN)], idx_win.at[b],
                              sems[b][0]).wait()
        pltpu.make_async_copy(val_hbm.at[pl.ds(woff, WIN)], val_win.at[b],
                              sems[b][1]).wait()

    def _flush_row(r, _):
        pltpu.sync_copy(st_val.at[r], chunk.at[st_idx.at[r]], add=True)
        for z in range(SROW // L):
            st_val[r, pl.ds(z * L, L)] = zf
        return 0

    def chunk_body(p, _):
        chunk_id = 2 * p + c
        lo = chunk_id * CH
        lo_vec = jnp.full((L,), lo, jnp.int32)
        live = chunk_id < NCHUNK

        @pl.when(live)
        def _process():
            pltpu.sync_copy(zeros_hbm, chunk.at[pl.ds(zslice, TSLICE)])
            plsc.subcore_barrier()

            def process_win(w, b, cnt_vec):
                _wait(w, b)

                def vec_body(i, cnt_vec):
                    iv = idx_win[b, pl.ds(i * L, L)]
                    vv = val_win[b, pl.ds(i * L, L)]
                    u = iv - lo_vec
                    m = plsc.bitcast(u, jnp.uint32) < ch_u
                    mi = jnp.where(m, one_i, zi)
                    pos = cnt_vec + plsc.cumsum(mi) - mi
                    plsc.store_scatter(st_idx, [pos >> 7, pos & (SROW - 1)],
                                       u, mask=m)
                    plsc.store_scatter(st_val, [pos >> 7, pos & (SROW - 1)],
                                       vv, mask=m)
                    return cnt_vec + plsc.all_reduce_population_count(m)

                cnt_vec = lax.fori_loop(0, NVEC, vec_body, cnt_vec)

                cnt = jnp.max(cnt_vec)
                rows = cnt >> 7
                lax.fori_loop(0, rows, _flush_row, 0)

                @pl.when(rows > 0)
                def _carry_partial():
                    for z in range(SROW // L):
                        st_idx[0, pl.ds(z * L, L)] = st_idx[rows,
                                                            pl.ds(z * L, L)]
                        st_val[0, pl.ds(z * L, L)] = st_val[rows,
                                                            pl.ds(z * L, L)]
                    for z in range(SROW // L):
                        st_val[rows, pl.ds(z * L, L)] = zf

                return jnp.full((L,), cnt & (SROW - 1), jnp.int32)

            _start(0, 0)

            def outer(k, cnt_vec):
                w0 = 2 * k
                _start(w0 + 1, 1)
                cnt_vec = process_win(w0, 0, cnt_vec)

                @pl.when(k < NWIN // 2 - 1)
                def _prefetch():
                    _start(w0 + 2, 0)

                return process_win(w0 + 1, 1, cnt_vec)

            cnt_vec = lax.fori_loop(0, NWIN // 2, outer, zi)

            cnt = jnp.max(cnt_vec)
            lax.fori_loop(0, (cnt + SROW - 1) >> 7, _flush_row, 0)

            plsc.subcore_barrier()
            pltpu.sync_copy(chunk.at[pl.ds(zslice, TSLICE)],
                            out_hbm.at[pl.ds(lo + zslice, TSLICE)])

        return 0

    lax.fori_loop(0, NCH_SC, chunk_body, 0)


_scatter_add = pl.kernel(
    _body,
    out_type=jax.ShapeDtypeStruct((LIN,), jnp.float32),
    mesh=plsc.VectorSubcoreMesh(core_axis_name="c", subcore_axis_name="s"),
    compiler_params=pltpu.CompilerParams(needs_layout_passes=False),
    scratch_types=[
        pltpu.VMEM((2, WIN), jnp.int32),
        pltpu.VMEM((2, WIN), jnp.float32),
        pltpu.VMEM((NROW, SROW), jnp.int32),
        pltpu.VMEM((NROW, SROW), jnp.float32),
        pltpu.VMEM_SHARED((CH,), jnp.float32),
        pltpu.SemaphoreType.DMA,
        pltpu.SemaphoreType.DMA,
        pltpu.SemaphoreType.DMA,
        pltpu.SemaphoreType.DMA,
    ],
)


@jax.jit
def kernel(x_in, x_ind_in):
    flat_idx = x_ind_in.reshape(-1)
    flat_val = x_in.reshape(-1)
    zeros = jnp.zeros((TSLICE,), jnp.float32)
    out = _scatter_add(flat_idx, flat_val, zeros)
    return out.reshape(OUT_SHAPE)


# speedup vs baseline: 11.1929x; 2.4840x over previous
import jax
import jax.numpy as jnp
from jax import lax
from jax.experimental import pallas as pl
from jax.experimental.pallas import tpu as pltpu
from jax.experimental.pallas import tpu_sc as plsc

N, H, W, C = 8, 112, 112, 96
OUT_SHAPE = (N, 2 * H, 2 * W, C)
LIN = N * 2 * H * 2 * W * C
NPAIRS = N * H * W * C

NC, NS, L = 2, 16, 16
CH = 1_835_008
NCHUNK = 21
NCH_SC = (NCHUNK + 1) // 2

PAIRS_PER_TILE = NPAIRS // NS
WIN = 2048
NWIN = PAIRS_PER_TILE // WIN
NVEC = WIN // L
SROW = 128
NROW = WIN // SROW + 2
TSLICE = CH // NS


def _body(idx_hbm, val_hbm, zeros_hbm, out_hbm, idx_win, val_win, st_idx,
          st_val, chunk, sem_i0, sem_v0, sem_i1, sem_v1):
    c = lax.axis_index("c")
    s = lax.axis_index("s")
    pair_base = s * PAIRS_PER_TILE
    zslice = s * TSLICE
    zf = jnp.zeros((L,), jnp.float32)
    zi = jnp.zeros((L,), jnp.int32)
    one_i = jnp.ones((L,), jnp.int32)
    ch_u = jnp.full((L,), CH, jnp.uint32)
    sems = ((sem_i0, sem_v0), (sem_i1, sem_v1))

    for r in range(NROW):
        for z in range(SROW // L):
            st_idx[r, pl.ds(z * L, L)] = zi
            st_val[r, pl.ds(z * L, L)] = zf

    def _start(w, b):
        woff = pair_base + w * WIN
        pltpu.async_copy(idx_hbm.at[pl.ds(woff, WIN)], idx_win.at[b],
                         sems[b][0])
        pltpu.async_copy(val_hbm.at[pl.ds(woff, WIN)], val_win.at[b],
                         sems[b][1])

    def _wait(w, b):
        woff = pair_base + w * WIN
        pltpu.make_async_copy(idx_hbm.at[pl.ds(woff, WIN)], idx_win.at[b],
                              sems[b][0]).wait()
        pltpu.make_async_copy(val_hbm.at[pl.ds(woff, WIN)], val_win.at[b],
                              sems[b][1]).wait()

    def _flush_row(r, _):
        pltpu.sync_copy(st_val.at[r], chunk.at[st_idx.at[r]], add=True)
        for z in range(SROW // L):
            st_val[r, pl.ds(z * L, L)] = zf
        return 0

    def chunk_body(p, _):
        chunk_id = 2 * p + c
        lo = chunk_id * CH
        lo_vec = jnp.full((L,), lo, jnp.int32)
        live = chunk_id < NCHUNK

        @pl.when(live)
        def _process():
            pltpu.sync_copy(zeros_hbm, chunk.at[pl.ds(zslice, TSLICE)])
            plsc.subcore_barrier()

            def process_win(w, b, cnt_vec):
                _wait(w, b)

                @plsc.parallel_loop(0, NVEC, 1, unroll=4, carry=cnt_vec)
                def vec_loop(i, cnt_vec):
                    iv = idx_win[b, pl.ds(i * L, L)]
                    vv = val_win[b, pl.ds(i * L, L)]
                    u = iv - lo_vec
                    m = plsc.bitcast(u, jnp.uint32) < ch_u
                    mi = jnp.where(m, one_i, zi)
                    pos = cnt_vec + plsc.cumsum(mi) - mi
                    plsc.store_scatter(st_idx, [pos >> 7, pos & (SROW - 1)],
                                       u, mask=m)
                    plsc.store_scatter(st_val, [pos >> 7, pos & (SROW - 1)],
                                       vv, mask=m)
                    return cnt_vec + plsc.all_reduce_population_count(m)

                cnt_vec = vec_loop

                cnt = jnp.max(cnt_vec)
                rows = cnt >> 7
                lax.fori_loop(0, rows, _flush_row, 0)

                @pl.when(rows > 0)
                def _carry_partial():
                    for z in range(SROW // L):
                        st_idx[0, pl.ds(z * L, L)] = st_idx[rows,
                                                            pl.ds(z * L, L)]
                        st_val[0, pl.ds(z * L, L)] = st_val[rows,
                                                            pl.ds(z * L, L)]
                    for z in range(SROW // L):
                        st_val[rows, pl.ds(z * L, L)] = zf

                return jnp.full((L,), cnt & (SROW - 1), jnp.int32)

            _start(0, 0)

            def outer(k, cnt_vec):
                w0 = 2 * k
                _start(w0 + 1, 1)
                cnt_vec = process_win(w0, 0, cnt_vec)

                @pl.when(k < NWIN // 2 - 1)
                def _prefetch():
                    _start(w0 + 2, 0)

                return process_win(w0 + 1, 1, cnt_vec)

            cnt_vec = lax.fori_loop(0, NWIN // 2, outer, zi)

            cnt = jnp.max(cnt_vec)
            lax.fori_loop(0, (cnt + SROW - 1) >> 7, _flush_row, 0)

            plsc.subcore_barrier()
            pltpu.sync_copy(chunk.at[pl.ds(zslice, TSLICE)],
                            out_hbm.at[pl.ds(lo + zslice, TSLICE)])

        return 0

    lax.fori_loop(0, NCH_SC, chunk_body, 0)


_scatter_add = pl.kernel(
    _body,
    out_type=jax.ShapeDtypeStruct((LIN,), jnp.float32),
    mesh=plsc.VectorSubcoreMesh(core_axis_name="c", subcore_axis_name="s"),
    compiler_params=pltpu.CompilerParams(needs_layout_passes=False),
    scratch_types=[
        pltpu.VMEM((2, WIN), jnp.int32),
        pltpu.VMEM((2, WIN), jnp.float32),
        pltpu.VMEM((NROW, SROW), jnp.int32),
        pltpu.VMEM((NROW, SROW), jnp.float32),
        pltpu.VMEM_SHARED((CH,), jnp.float32),
        pltpu.SemaphoreType.DMA,
        pltpu.SemaphoreType.DMA,
        pltpu.SemaphoreType.DMA,
        pltpu.SemaphoreType.DMA,
    ],
)


@jax.jit
def kernel(x_in, x_ind_in):
    flat_idx = x_ind_in.reshape(-1)
    flat_val = x_in.reshape(-1)
    zeros = jnp.zeros((TSLICE,), jnp.float32)
    out = _scatter_add(flat_idx, flat_val, zeros)
    return out.reshape(OUT_SHAPE)


# speedup vs baseline: 11.4164x; 1.0200x over previous
import jax
import jax.numpy as jnp
from jax import lax
from jax.experimental import pallas as pl
from jax.experimental.pallas import tpu as pltpu
from jax.experimental.pallas import tpu_sc as plsc

N, H, W, C = 8, 112, 112, 96
OUT_SHAPE = (N, 2 * H, 2 * W, C)
LIN = N * 2 * H * 2 * W * C
NPAIRS = N * H * W * C

NC, NS, L = 2, 16, 16
CH = 1_835_008
NCHUNK = 21
NCH_SC = (NCHUNK + 1) // 2

PAIRS_PER_TILE = NPAIRS // NS
WIN = 2048
NWIN = PAIRS_PER_TILE // WIN
NVEC = WIN // L
SROW = 128
NROW = WIN // SROW + 2
TSLICE = CH // NS


def _body(idx_hbm, val_hbm, zeros_hbm, out_hbm, idx_win, val_win, st_idx,
          st_val, chunk, sem_i0, sem_v0, sem_i1, sem_v1):
    c = lax.axis_index("c")
    s = lax.axis_index("s")
    pair_base = s * PAIRS_PER_TILE
    zslice = s * TSLICE
    zf = jnp.zeros((L,), jnp.float32)
    zi = jnp.zeros((L,), jnp.int32)
    one_i = jnp.ones((L,), jnp.int32)
    ch_u = jnp.full((L,), CH, jnp.uint32)
    sems = ((sem_i0, sem_v0), (sem_i1, sem_v1))

    for r in range(NROW):
        for z in range(SROW // L):
            st_idx[r, pl.ds(z * L, L)] = zi
            st_val[r, pl.ds(z * L, L)] = zf

    def _start(w, b):
        woff = pair_base + w * WIN
        pltpu.async_copy(idx_hbm.at[pl.ds(woff, WIN)], idx_win.at[b],
                         sems[b][0])
        pltpu.async_copy(val_hbm.at[pl.ds(woff, WIN)], val_win.at[b],
                         sems[b][1])

    def _wait(w, b):
        woff = pair_base + w * WIN
        pltpu.make_async_copy(idx_hbm.at[pl.ds(woff, WIN)], idx_win.at[b],
                              sems[b][0]).wait()
        pltpu.make_async_copy(val_hbm.at[pl.ds(woff, WIN)], val_win.at[b],
                              sems[b][1]).wait()

    def _flush_row(r, _):
        pltpu.sync_copy(st_val.at[r], chunk.at[st_idx.at[r]], add=True)
        for z in range(SROW // L):
            st_val[r, pl.ds(z * L, L)] = zf
        return 0

    def chunk_body(p, _):
        chunk_id = 2 * p + c
        lo = chunk_id * CH
        lo_vec = jnp.full((L,), lo, jnp.int32)
        live = chunk_id < NCHUNK

        @pl.when(live)
        def _process():
            pltpu.sync_copy(zeros_hbm, chunk.at[pl.ds(zslice, TSLICE)])
            plsc.subcore_barrier()

            def process_win(w, b, cnt_vec):
                _wait(w, b)

                @plsc.parallel_loop(0, NVEC, 1, unroll=8, carry=cnt_vec)
                def vec_loop(i, cnt_vec):
                    iv = idx_win[b, pl.ds(i * L, L)]
                    vv = val_win[b, pl.ds(i * L, L)]
                    u = iv - lo_vec
                    m = plsc.bitcast(u, jnp.uint32) < ch_u
                    mi = jnp.where(m, one_i, zi)
                    pos = cnt_vec + plsc.cumsum(mi) - mi
                    plsc.store_scatter(st_idx, [pos >> 7, pos & (SROW - 1)],
                                       u, mask=m)
                    plsc.store_scatter(st_val, [pos >> 7, pos & (SROW - 1)],
                                       vv, mask=m)
                    return cnt_vec + plsc.all_reduce_population_count(m)

                cnt_vec = vec_loop

                cnt = jnp.max(cnt_vec)
                rows = cnt >> 7
                lax.fori_loop(0, rows, _flush_row, 0)

                @pl.when(rows > 0)
                def _carry_partial():
                    for z in range(SROW // L):
                        st_idx[0, pl.ds(z * L, L)] = st_idx[rows,
                                                            pl.ds(z * L, L)]
                        st_val[0, pl.ds(z * L, L)] = st_val[rows,
                                                            pl.ds(z * L, L)]
                    for z in range(SROW // L):
                        st_val[rows, pl.ds(z * L, L)] = zf

                return jnp.full((L,), cnt & (SROW - 1), jnp.int32)

            _start(0, 0)

            def outer(k, cnt_vec):
                w0 = 2 * k
                _start(w0 + 1, 1)
                cnt_vec = process_win(w0, 0, cnt_vec)

                @pl.when(k < NWIN // 2 - 1)
                def _prefetch():
                    _start(w0 + 2, 0)

                return process_win(w0 + 1, 1, cnt_vec)

            cnt_vec = lax.fori_loop(0, NWIN // 2, outer, zi)

            cnt = jnp.max(cnt_vec)
            lax.fori_loop(0, (cnt + SROW - 1) >> 7, _flush_row, 0)

            plsc.subcore_barrier()
            pltpu.sync_copy(chunk.at[pl.ds(zslice, TSLICE)],
                            out_hbm.at[pl.ds(lo + zslice, TSLICE)])

        return 0

    lax.fori_loop(0, NCH_SC, chunk_body, 0)


_scatter_add = pl.kernel(
    _body,
    out_type=jax.ShapeDtypeStruct((LIN,), jnp.float32),
    mesh=plsc.VectorSubcoreMesh(core_axis_name="c", subcore_axis_name="s"),
    compiler_params=pltpu.CompilerParams(needs_layout_passes=False),
    scratch_types=[
        pltpu.VMEM((2, WIN), jnp.int32),
        pltpu.VMEM((2, WIN), jnp.float32),
        pltpu.VMEM((NROW, SROW), jnp.int32),
        pltpu.VMEM((NROW, SROW), jnp.float32),
        pltpu.VMEM_SHARED((CH,), jnp.float32),
        pltpu.SemaphoreType.DMA,
        pltpu.SemaphoreType.DMA,
        pltpu.SemaphoreType.DMA,
        pltpu.SemaphoreType.DMA,
    ],
)


@jax.jit
def kernel(x_in, x_ind_in):
    flat_idx = x_ind_in.reshape(-1)
    flat_val = x_in.reshape(-1)
    zeros = jnp.zeros((TSLICE,), jnp.float32)
    out = _scatter_add(flat_idx, flat_val, zeros)
    return out.reshape(OUT_SHAPE)


# speedup vs baseline: 14.4485x; 1.2656x over previous
import jax
import jax.numpy as jnp
from jax import lax
from jax.experimental import pallas as pl
from jax.experimental.pallas import tpu as pltpu
from jax.experimental.pallas import tpu_sc as plsc

N, H, W, C = 8, 112, 112, 96
OUT_SHAPE = (N, 2 * H, 2 * W, C)
LIN = N * 2 * H * 2 * W * C
NPAIRS = N * H * W * C

NC, NS, L = 2, 16, 16
CH = 1_835_008
NCHUNK = 21
NCH0 = 10
B0 = NCH0 * CH
NCH_TRIPS = NCHUNK - NCH0

PAIRS_PER_TILE = NPAIRS // NS
WIN = 2048
WROW = WIN // 128
NWIN = PAIRS_PER_TILE // WIN
NVEC = WIN // L
SROW = 128
NROW = 24
CAPR = PAIRS_PER_TILE // 128 + 32
TSLICE = CH // NS


def _body(idxr_hbm, valr_hbm, zeros_hbm, out_hbm, sidx_hbm, sval_hbm,
          idx_win, val_win, st_idx, st_val, chunk,
          sem_i0, sem_v0, sem_i1, sem_v1):
    c = lax.axis_index("c")
    s = lax.axis_index("s")
    row_base = s * (PAIRS_PER_TILE // 128)
    zslice = s * TSLICE
    zf = jnp.zeros((L,), jnp.float32)
    zi = jnp.zeros((L,), jnp.int32)
    one_i = jnp.ones((L,), jnp.int32)
    ch_u = jnp.full((L,), CH, jnp.uint32)
    half_lo = c * B0
    half_lo_vec = jnp.full((L,), half_lo, jnp.int32)
    half_sz = jnp.where(c == 0, B0, LIN - B0)
    half_sz_u = plsc.bitcast(jnp.full((L,), half_sz, jnp.int32), jnp.uint32)
    sems = ((sem_i0, sem_v0), (sem_i1, sem_v1))

    def _ld(ref, b, i):
        return ref[b, i >> 3, pl.ds((i & 7) * L, L)]

    def _start0(w, b):
        pltpu.async_copy(idxr_hbm.at[pl.ds(row_base + w * WROW, WROW)],
                         idx_win.at[b], sems[b][0])
        pltpu.async_copy(valr_hbm.at[pl.ds(row_base + w * WROW, WROW)],
                         val_win.at[b], sems[b][1])

    def _wait0(w, b):
        pltpu.make_async_copy(idxr_hbm.at[pl.ds(row_base + w * WROW, WROW)],
                              idx_win.at[b], sems[b][0]).wait()
        pltpu.make_async_copy(valr_hbm.at[pl.ds(row_base + w * WROW, WROW)],
                              val_win.at[b], sems[b][1]).wait()

    def _flush_group(g, roff):
        ro = pl.multiple_of(roff, 8)
        pltpu.sync_copy(st_idx.at[pl.ds(g * 8, 8)],
                        sidx_hbm.at[c, s, pl.ds(ro, 8)])
        pltpu.sync_copy(st_val.at[pl.ds(g * 8, 8)],
                        sval_hbm.at[c, s, pl.ds(ro, 8)])
        return roff + 8

    def process0(w, b, carry):
        cnt_vec, roff = carry
        _wait0(w, b)

        @plsc.parallel_loop(0, NVEC, 1, unroll=4, carry=cnt_vec)
        def vec_loop(i, cnt_vec):
            iv = _ld(idx_win, b, i)
            vv = _ld(val_win, b, i)
            u = iv - half_lo_vec
            m = plsc.bitcast(u, jnp.uint32) < half_sz_u
            mi = jnp.where(m, one_i, zi)
            pos = cnt_vec + plsc.cumsum(mi) - mi
            plsc.store_scatter(st_idx, [pos >> 7, pos & (SROW - 1)], iv,
                               mask=m)
            plsc.store_scatter(st_val, [pos >> 7, pos & (SROW - 1)], vv,
                               mask=m)
            return cnt_vec + plsc.all_reduce_population_count(m)

        cnt_vec = vec_loop
        cnt = jnp.max(cnt_vec)
        fg = cnt >> 10
        roff = lax.fori_loop(0, fg, _flush_group, roff)

        @pl.when(fg > 0)
        def _carry_rem():
            rb = fg * 8
            for rr in range(8):
                for z in range(SROW // L):
                    st_idx[rr, pl.ds(z * L, L)] = st_idx[rb + rr,
                                                         pl.ds(z * L, L)]
                    st_val[rr, pl.ds(z * L, L)] = st_val[rb + rr,
                                                         pl.ds(z * L, L)]

        return jnp.full((L,), cnt & 1023, jnp.int32), roff

    _start0(0, 0)

    def outer0(k, carry):
        w0 = 2 * k
        _start0(w0 + 1, 1)
        carry = process0(w0, 0, carry)

        @pl.when(k < NWIN // 2 - 1)
        def _prefetch0():
            _start0(w0 + 2, 0)

        return process0(w0 + 1, 1, carry)

    cnt_vec, roff = lax.fori_loop(0, NWIN // 2, outer0, (zi, jnp.int32(0)))

    cnt = jnp.max(cnt_vec)
    iota = lax.iota(jnp.int32, L)
    for j in range(8):
        posz = jnp.minimum(cnt + j * L + iota, NROW * SROW - 1)
        plsc.store_scatter(st_val, [posz >> 7, posz & (SROW - 1)], zf)

    def _zrow(r, x):
        for z in range(SROW // L):
            st_val[r, pl.ds(z * L, L)] = zf
        return x

    fin_g = (cnt + 1023) >> 10
    lax.fori_loop(jnp.minimum((cnt >> 7) + 1, fin_g * 8), fin_g * 8, _zrow, 0)
    roff = lax.fori_loop(0, fin_g, _flush_group, roff)

    for r in range(8):
        for z in range(SROW // L):
            st_idx[r, pl.ds(z * L, L)] = zi
            st_val[r, pl.ds(z * L, L)] = zf

    def _padg(g, roff):
        return _flush_group(0, roff)

    roff = lax.fori_loop(0, ((-roff) & 31) >> 3, _padg, roff)
    nk = roff >> 5

    for r in range(NROW):
        for z in range(SROW // L):
            st_idx[r, pl.ds(z * L, L)] = zi
            st_val[r, pl.ds(z * L, L)] = zf

    def _start1(w, b):
        wo = pl.multiple_of(w * WROW, 8)
        pltpu.async_copy(sidx_hbm.at[c, s, pl.ds(wo, WROW)],
                         idx_win.at[b], sems[b][0])
        pltpu.async_copy(sval_hbm.at[c, s, pl.ds(wo, WROW)],
                         val_win.at[b], sems[b][1])

    def _wait1(w, b):
        wo = pl.multiple_of(w * WROW, 8)
        pltpu.make_async_copy(sidx_hbm.at[c, s, pl.ds(wo, WROW)],
                              idx_win.at[b], sems[b][0]).wait()
        pltpu.make_async_copy(sval_hbm.at[c, s, pl.ds(wo, WROW)],
                              val_win.at[b], sems[b][1]).wait()

    def _flush_row(r, x):
        pltpu.sync_copy(st_val.at[r], chunk.at[st_idx.at[r]], add=True)
        for z in range(SROW // L):
            st_val[r, pl.ds(z * L, L)] = zf
        return x

    def chunk_body(p, _):
        chunk_id = c * NCH0 + p
        lo = chunk_id * CH
        lo_vec = jnp.full((L,), lo, jnp.int32)
        live = jnp.logical_or(c == 1, p < NCH0)

        @pl.when(live)
        def _process():
            pltpu.sync_copy(zeros_hbm, chunk.at[pl.ds(zslice, TSLICE)])
            plsc.subcore_barrier()

            def process1(w, b, cnt_vec):
                _wait1(w, b)

                @plsc.parallel_loop(0, NVEC, 1, unroll=4, carry=cnt_vec)
                def vec_loop(i, cnt_vec):
                    iv = _ld(idx_win, b, i)
                    vv = _ld(val_win, b, i)
                    u = iv - lo_vec
                    m = plsc.bitcast(u, jnp.uint32) < ch_u
                    mi = jnp.where(m, one_i, zi)
                    pos = cnt_vec + plsc.cumsum(mi) - mi
                    plsc.store_scatter(st_idx, [pos >> 7, pos & (SROW - 1)],
                                       u, mask=m)
                    plsc.store_scatter(st_val, [pos >> 7, pos & (SROW - 1)],
                                       vv, mask=m)
                    return cnt_vec + plsc.all_reduce_population_count(m)

                cnt_vec = vec_loop

                cnt = jnp.max(cnt_vec)
                rows = cnt >> 7
                lax.fori_loop(0, rows, _flush_row, 0)

                @pl.when(rows > 0)
                def _carry_partial():
                    for z in range(SROW // L):
                        st_idx[0, pl.ds(z * L, L)] = st_idx[rows,
                                                            pl.ds(z * L, L)]
                        st_val[0, pl.ds(z * L, L)] = st_val[rows,
                                                            pl.ds(z * L, L)]
                    for z in range(SROW // L):
                        st_val[rows, pl.ds(z * L, L)] = zf

                return jnp.full((L,), cnt & (SROW - 1), jnp.int32)

            @pl.when(nk > 0)
            def _prime():
                _start1(0, 0)

            def outer1(k, cnt_vec):
                w0 = 2 * k
                _start1(w0 + 1, 1)
                cnt_vec = process1(w0, 0, cnt_vec)

                @pl.when(k < nk - 1)
                def _prefetch1():
                    _start1(w0 + 2, 0)

                return process1(w0 + 1, 1, cnt_vec)

            cnt_vec = lax.fori_loop(0, nk, outer1, zi)

            cnt = jnp.max(cnt_vec)
            lax.fori_loop(0, (cnt + SROW - 1) >> 7, _flush_row, 0)

            plsc.subcore_barrier()
            pltpu.sync_copy(chunk.at[pl.ds(zslice, TSLICE)],
                            out_hbm.at[pl.ds(lo + zslice, TSLICE)])

        return 0

    lax.fori_loop(0, NCH_TRIPS, chunk_body, 0)


_scatter_add = pl.kernel(
    _body,
    out_type=(
        jax.ShapeDtypeStruct((LIN,), jnp.float32),
        jax.ShapeDtypeStruct((NC, NS, CAPR, 128), jnp.int32),
        jax.ShapeDtypeStruct((NC, NS, CAPR, 128), jnp.float32),
    ),
    mesh=plsc.VectorSubcoreMesh(core_axis_name="c", subcore_axis_name="s"),
    compiler_params=pltpu.CompilerParams(needs_layout_passes=False),
    scratch_types=[
        pltpu.VMEM((2, WROW, 128), jnp.int32),
        pltpu.VMEM((2, WROW, 128), jnp.float32),
        pltpu.VMEM((NROW, SROW), jnp.int32),
        pltpu.VMEM((NROW, SROW), jnp.float32),
        pltpu.VMEM_SHARED((CH,), jnp.float32),
        pltpu.SemaphoreType.DMA,
        pltpu.SemaphoreType.DMA,
        pltpu.SemaphoreType.DMA,
        pltpu.SemaphoreType.DMA,
    ],
)


@jax.jit
def kernel(x_in, x_ind_in):
    flat_idx = x_ind_in.reshape(-1, 128)
    flat_val = x_in.reshape(-1, 128)
    zeros = jnp.zeros((TSLICE,), jnp.float32)
    out, _, _ = _scatter_add(flat_idx, flat_val, zeros)
    return out.reshape(OUT_SHAPE)
